# Initial kernel scaffold; baseline (speedup 1.0000x reference)
import jax
import jax.numpy as jnp
from jax.experimental import pallas as pl


def kernel(codes, emb, W1, U1, b1, W2, U2, b2, Wout, bout):
    raise NotImplementedError("write your pallas kernel here")


# speedup vs baseline: 6.4457x; 6.4457x over previous
import functools

import jax
import jax.numpy as jnp
from jax.experimental import pallas as pl
from jax.experimental.pallas import tpu as pltpu

CHUNK = 16


def _gates(z, c, units):
    i = 0.5 * jnp.tanh(0.5 * z[:, :units]) + 0.5
    f = 0.5 * jnp.tanh(0.5 * z[:, units:2 * units]) + 0.5
    g = jnp.tanh(z[:, 2 * units:3 * units])
    o = 0.5 * jnp.tanh(0.5 * z[:, 3 * units:]) + 0.5
    c_new = f * c + i * g
    h_new = o * jnp.tanh(c_new)
    return h_new, c_new


def _body(codes_ref, emb_ref, w1_ref, b1_ref, u1_ref,
          w2_ref, b2_ref, u2_ref, wout_ref, bout_ref, out_ref,
          z1_scr, z2_scr, h1_scr, hs_scr,
          h1c_scr, c1c_scr, h2c_scr, c2c_scr,
          *, bsz, nc, units):
    k = pl.program_id(0)

    @pl.when(k == 0)
    def _init():
        h1c_scr[...] = jnp.zeros_like(h1c_scr)
        c1c_scr[...] = jnp.zeros_like(c1c_scr)
        h2c_scr[...] = jnp.zeros_like(h2c_scr)
        c2c_scr[...] = jnp.zeros_like(c2c_scr)

    hin = h1_scr[...].reshape(CHUNK * bsz, units).astype(jnp.bfloat16)
    z2_scr[...] = (jnp.dot(hin, w2_ref[...], preferred_element_type=jnp.float32)
                   + b2_ref[...]).reshape(CHUNK, bsz, 4 * units)

    codes = codes_ref[...]
    onehot = (codes[:, :, None]
              == jax.lax.broadcasted_iota(jnp.int32, (CHUNK, bsz, nc), 2)
              ).astype(jnp.float32)
    x = jnp.dot(onehot.reshape(CHUNK * bsz, nc), emb_ref[...],
                preferred_element_type=jnp.float32)
    z1_scr[...] = (jnp.dot(x, w1_ref[...], preferred_element_type=jnp.float32)
                   + b1_ref[...]).reshape(CHUNK, bsz, 4 * units)

    h1 = h1c_scr[...]
    c1 = c1c_scr[...]
    h2 = h2c_scr[...]
    c2 = c2c_scr[...]
    u1 = u1_ref[...]
    u2 = u2_ref[...]
    for j in range(CHUNK):
        za = z1_scr[j] + jnp.dot(h1.astype(jnp.bfloat16), u1,
                                 preferred_element_type=jnp.float32)
        h1, c1 = _gates(za, c1, units)
        h1_scr[j] = h1
        zb = z2_scr[j] + jnp.dot(h2.astype(jnp.bfloat16), u2,
                                 preferred_element_type=jnp.float32)
        h2, c2 = _gates(zb, c2, units)
        hs_scr[j] = h2
    h1c_scr[...] = h1
    c1c_scr[...] = c1

    @pl.when(k > 0)
    def _commit2():
        h2c_scr[...] = h2
        c2c_scr[...] = c2

    hflat = hs_scr[...].reshape(CHUNK * bsz, units).astype(jnp.bfloat16)
    out_ref[...] = (jnp.dot(hflat, wout_ref[...],
                            preferred_element_type=jnp.float32)
                    + bout_ref[...]).reshape(CHUNK, bsz, nc)


def kernel(codes, emb, W1, U1, b1, W2, U2, b2, Wout, bout):
    bsz, t = codes.shape
    nc, emb_dim = emb.shape
    units = U1.shape[0]
    h4 = 4 * units
    nblk = t // CHUNK

    codes_t = codes.T
    b1r = b1.reshape(1, h4)
    b2r = b2.reshape(1, h4)
    boutr = bout.reshape(1, nc)
    U1b = U1.astype(jnp.bfloat16)
    U2b = U2.astype(jnp.bfloat16)
    W2b = W2.astype(jnp.bfloat16)
    Woutb = Wout.astype(jnp.bfloat16)

    fixed = lambda i: (0, 0)
    logits_t = pl.pallas_call(
        functools.partial(_body, bsz=bsz, nc=nc, units=units),
        grid=(nblk + 1,),
        in_specs=[
            pl.BlockSpec((CHUNK, bsz), lambda i: (jnp.minimum(i, nblk - 1), 0)),
            pl.BlockSpec((nc, emb_dim), fixed),
            pl.BlockSpec((emb_dim, h4), fixed),
            pl.BlockSpec((1, h4), fixed),
            pl.BlockSpec((units, h4), fixed),
            pl.BlockSpec((units, h4), fixed),
            pl.BlockSpec((1, h4), fixed),
            pl.BlockSpec((units, h4), fixed),
            pl.BlockSpec((units, nc), fixed),
            pl.BlockSpec((1, nc), fixed),
        ],
        out_specs=pl.BlockSpec((CHUNK, bsz, nc),
                               lambda i: (jnp.maximum(i - 1, 0), 0, 0)),
        out_shape=jax.ShapeDtypeStruct((t, bsz, nc), jnp.float32),
        scratch_shapes=[
            pltpu.VMEM((CHUNK, bsz, h4), jnp.float32),
            pltpu.VMEM((CHUNK, bsz, h4), jnp.float32),
            pltpu.VMEM((CHUNK, bsz, units), jnp.float32),
            pltpu.VMEM((CHUNK, bsz, units), jnp.float32),
            pltpu.VMEM((bsz, units), jnp.float32),
            pltpu.VMEM((bsz, units), jnp.float32),
            pltpu.VMEM((bsz, units), jnp.float32),
            pltpu.VMEM((bsz, units), jnp.float32),
        ],
    )(codes_t, emb, W1, b1r, U1b, W2b, b2r, U2b, Woutb, boutr)

    return jnp.swapaxes(logits_t, 0, 1)


# speedup vs baseline: 6.4477x; 1.0003x over previous
import functools

import jax
import jax.numpy as jnp
from jax.experimental import pallas as pl
from jax.experimental.pallas import tpu as pltpu

CHUNK = 16


def _gates(z, c, units):
    i = jax.nn.sigmoid(z[:, :units])
    f = jax.nn.sigmoid(z[:, units:2 * units])
    g = jnp.tanh(z[:, 2 * units:3 * units])
    o = jax.nn.sigmoid(z[:, 3 * units:])
    c_new = f * c + i * g
    h_new = o * jnp.tanh(c_new)
    return h_new, c_new


def _body(codes_ref, emb_ref, w1_ref, b1_ref, u1_ref,
          w2_ref, b2_ref, u2_ref, wout_ref, bout_ref, out_ref,
          z1_scr, z2_scr, h1_scr, hs_scr,
          h1c_scr, c1c_scr, h2c_scr, c2c_scr,
          *, bsz, nc, units):
    k = pl.program_id(0)

    @pl.when(k == 0)
    def _init():
        h1c_scr[...] = jnp.zeros_like(h1c_scr)
        c1c_scr[...] = jnp.zeros_like(c1c_scr)
        h2c_scr[...] = jnp.zeros_like(h2c_scr)
        c2c_scr[...] = jnp.zeros_like(c2c_scr)

    hin = h1_scr[...].reshape(CHUNK * bsz, units).astype(jnp.bfloat16)
    z2_scr[...] = (jnp.dot(hin, w2_ref[...], preferred_element_type=jnp.float32)
                   + b2_ref[...]).reshape(CHUNK, bsz, 4 * units)

    codes = codes_ref[...]
    onehot = (codes[:, :, None]
              == jax.lax.broadcasted_iota(jnp.int32, (CHUNK, bsz, nc), 2)
              ).astype(jnp.float32)
    x = jnp.dot(onehot.reshape(CHUNK * bsz, nc), emb_ref[...],
                preferred_element_type=jnp.float32)
    z1_scr[...] = (jnp.dot(x, w1_ref[...], preferred_element_type=jnp.float32)
                   + b1_ref[...]).reshape(CHUNK, bsz, 4 * units)

    h1 = h1c_scr[...]
    c1 = c1c_scr[...]
    h2 = h2c_scr[...]
    c2 = c2c_scr[...]
    u1 = u1_ref[...]
    u2 = u2_ref[...]
    for j in range(CHUNK):
        za = z1_scr[j] + jnp.dot(h1.astype(jnp.bfloat16), u1,
                                 preferred_element_type=jnp.float32)
        h1, c1 = _gates(za, c1, units)
        h1_scr[j] = h1
        zb = z2_scr[j] + jnp.dot(h2.astype(jnp.bfloat16), u2,
                                 preferred_element_type=jnp.float32)
        h2, c2 = _gates(zb, c2, units)
        hs_scr[j] = h2
    h1c_scr[...] = h1
    c1c_scr[...] = c1

    @pl.when(k > 0)
    def _commit2():
        h2c_scr[...] = h2
        c2c_scr[...] = c2

    hflat = hs_scr[...].reshape(CHUNK * bsz, units).astype(jnp.bfloat16)
    out_ref[...] = (jnp.dot(hflat, wout_ref[...],
                            preferred_element_type=jnp.float32)
                    + bout_ref[...]).reshape(CHUNK, bsz, nc)


def kernel(codes, emb, W1, U1, b1, W2, U2, b2, Wout, bout):
    bsz, t = codes.shape
    nc, emb_dim = emb.shape
    units = U1.shape[0]
    h4 = 4 * units
    nblk = t // CHUNK

    codes_t = codes.T
    b1r = b1.reshape(1, h4)
    b2r = b2.reshape(1, h4)
    boutr = bout.reshape(1, nc)
    U1b = U1.astype(jnp.bfloat16)
    U2b = U2.astype(jnp.bfloat16)
    W2b = W2.astype(jnp.bfloat16)
    Woutb = Wout.astype(jnp.bfloat16)

    fixed = lambda i: (0, 0)
    logits_t = pl.pallas_call(
        functools.partial(_body, bsz=bsz, nc=nc, units=units),
        grid=(nblk + 1,),
        in_specs=[
            pl.BlockSpec((CHUNK, bsz), lambda i: (jnp.minimum(i, nblk - 1), 0)),
            pl.BlockSpec((nc, emb_dim), fixed),
            pl.BlockSpec((emb_dim, h4), fixed),
            pl.BlockSpec((1, h4), fixed),
            pl.BlockSpec((units, h4), fixed),
            pl.BlockSpec((units, h4), fixed),
            pl.BlockSpec((1, h4), fixed),
            pl.BlockSpec((units, h4), fixed),
            pl.BlockSpec((units, nc), fixed),
            pl.BlockSpec((1, nc), fixed),
        ],
        out_specs=pl.BlockSpec((CHUNK, bsz, nc),
                               lambda i: (jnp.maximum(i - 1, 0), 0, 0)),
        out_shape=jax.ShapeDtypeStruct((t, bsz, nc), jnp.float32),
        scratch_shapes=[
            pltpu.VMEM((CHUNK, bsz, h4), jnp.float32),
            pltpu.VMEM((CHUNK, bsz, h4), jnp.float32),
            pltpu.VMEM((CHUNK, bsz, units), jnp.float32),
            pltpu.VMEM((CHUNK, bsz, units), jnp.float32),
            pltpu.VMEM((bsz, units), jnp.float32),
            pltpu.VMEM((bsz, units), jnp.float32),
            pltpu.VMEM((bsz, units), jnp.float32),
            pltpu.VMEM((bsz, units), jnp.float32),
        ],
    )(codes_t, emb, W1, b1r, U1b, W2b, b2r, U2b, Woutb, boutr)

    return jnp.swapaxes(logits_t, 0, 1)
